# Initial kernel scaffold; baseline (speedup 1.0000x reference)
import jax
import jax.numpy as jnp
from jax.experimental import pallas as pl


def kernel(query_emb, index, k):
    raise NotImplementedError("write your pallas kernel here")


# speedup vs baseline: 16.8247x; 16.8247x over previous
import jax
import jax.numpy as jnp
from jax import lax
from jax.experimental import pallas as pl
from jax.experimental.pallas import tpu as pltpu

N = 1_000_000
D = 64
CHUNK = 8192
NCHUNK = (N + CHUNK - 1) // CHUNK
ROWS = NCHUNK * (CHUNK // 128)
K = 100
NSEL = 128
NEG = float("-inf")
IBIG = 2**31 - 1


def _topk_body(q_ref, x_ref, vals_ref, ids_ref, scores_ref, gmax_ref):
    c = pl.program_id(0)

    @pl.when(c == 0)
    def _init():
        gmax_ref[...] = jnp.full((128, 128), NEG, jnp.float32)

    x = x_ref[...]
    q = q_ref[...]
    s = lax.dot_general(q, x, (((1,), (1,)), ((), ())),
                        preferred_element_type=jnp.float32)
    base = c * CHUNK
    eidx = base + lax.broadcasted_iota(jnp.int32, (1, CHUNK), 1)
    s = jnp.where(eidx < N, s, NEG)
    s2 = s.reshape(CHUNK // 128, 128)
    scores_ref[pl.ds(c * (CHUNK // 128), CHUNK // 128), :] = s2
    gmax_ref[pl.ds(c, 1), :] = jnp.max(s2, axis=0, keepdims=True)

    @pl.when(c == NCHUNK - 1)
    def _select():
        gm = gmax_ref[...]
        g_iota = (lax.broadcasted_iota(jnp.int32, (128, 128), 0) * 128
                  + lax.broadcasted_iota(jnp.int32, (128, 128), 1))
        lane128 = lax.broadcasted_iota(jnp.int32, (1, 128), 1)
        row64 = lax.broadcasted_iota(jnp.int32, (64, 1), 0)

        def gather_one(i, carry):
            gm, cand, cidx = carry
            m = jnp.max(gm)
            g = jnp.min(jnp.where(gm == m, g_iota, IBIG))
            cg = g // 128
            j = g - cg * 128
            block = scores_ref[pl.ds(cg * 64, 64), :]
            col = jnp.sum(jnp.where(lane128 == j, block, 0.0),
                          axis=1, keepdims=True)
            ge = (cg * CHUNK + row64 * 128 + j).astype(jnp.int32)
            sel = lane128 == i
            cand = jnp.where(sel, col, cand)
            cidx = jnp.where(sel, ge, cidx)
            gm = jnp.where(g_iota == g, NEG, gm)
            return gm, cand, cidx

        cand0 = jnp.full((64, 128), NEG, jnp.float32)
        cidx0 = jnp.full((64, 128), IBIG, jnp.int32)
        _, cand, cidx = lax.fori_loop(0, NSEL, gather_one,
                                      (gmax_ref[...], cand0, cidx0))

        def pick_one(i, carry):
            cand, vals, ids = carry
            m = jnp.max(cand)
            idx = jnp.min(jnp.where(cand == m, cidx, IBIG))
            sel = lane128 == i
            vals = jnp.where(sel, m, vals)
            ids = jnp.where(sel, idx, ids)
            cand = jnp.where(cidx == idx, NEG, cand)
            return cand, vals, ids

        vals0 = jnp.zeros((1, 128), jnp.float32)
        ids0 = jnp.zeros((1, 128), jnp.int32)
        _, vals, ids = lax.fori_loop(0, K, pick_one, (cand, vals0, ids0))
        vals_ref[...] = vals
        ids_ref[...] = ids


def kernel(query_emb, index, k):
    del k
    vals, ids = pl.pallas_call(
        lambda q, x, v, i, s, g: _topk_body(q, x, v, i, s, g),
        grid=(NCHUNK,),
        in_specs=[
            pl.BlockSpec((1, D), lambda c: (0, 0)),
            pl.BlockSpec((CHUNK, D), lambda c: (c, 0)),
        ],
        out_specs=[
            pl.BlockSpec((1, 128), lambda c: (0, 0)),
            pl.BlockSpec((1, 128), lambda c: (0, 0)),
        ],
        out_shape=[
            jax.ShapeDtypeStruct((1, 128), jnp.float32),
            jax.ShapeDtypeStruct((1, 128), jnp.int32),
        ],
        scratch_shapes=[
            pltpu.VMEM((NCHUNK * (CHUNK // 128), 128), jnp.float32),
            pltpu.VMEM((128, 128), jnp.float32),
        ],
    )(query_emb, index)
    return ids[0, :K], vals[0, :K]


# speedup vs baseline: 17.3545x; 1.0315x over previous
import jax
import jax.numpy as jnp
from jax import lax
from jax.experimental import pallas as pl
from jax.experimental.pallas import tpu as pltpu

N = 1_000_000
D = 64
CHUNK = 32768
NCHUNK = (N + CHUNK - 1) // CHUNK
ROWS = NCHUNK * (CHUNK // 128)
K = 100
NSEL = 128
NEG = float("-inf")
IBIG = 2**31 - 1


def _topk_body(q_ref, x_ref, vals_ref, ids_ref, scores_ref, gmax_ref):
    c = pl.program_id(0)

    @pl.when(c == 0)
    def _init():
        gmax_ref[...] = jnp.full((128, 128), NEG, jnp.float32)

    x = x_ref[...]
    q = q_ref[...]
    s = lax.dot_general(q, x, (((1,), (1,)), ((), ())),
                        preferred_element_type=jnp.float32)
    base = c * CHUNK
    eidx = base + lax.broadcasted_iota(jnp.int32, (1, CHUNK), 1)
    s = jnp.where(eidx < N, s, NEG)
    s2 = s.reshape(CHUNK // 128, 128)
    scores_ref[pl.ds(c * (CHUNK // 128), CHUNK // 128), :] = s2
    gmax_ref[pl.ds(c, 1), :] = jnp.max(s2, axis=0, keepdims=True)

    @pl.when(c == NCHUNK - 1)
    def _select():
        gm = gmax_ref[...]
        g_iota = (lax.broadcasted_iota(jnp.int32, (128, 128), 0) * 128
                  + lax.broadcasted_iota(jnp.int32, (128, 128), 1))
        lane128 = lax.broadcasted_iota(jnp.int32, (1, 128), 1)
        rows = CHUNK // 128
        rowi = lax.broadcasted_iota(jnp.int32, (rows, 1), 0)

        def gather_one(i, carry):
            gm, cand, cidx = carry
            m = jnp.max(gm)
            g = jnp.min(jnp.where(gm == m, g_iota, IBIG))
            cg = g // 128
            j = g - cg * 128
            block = scores_ref[pl.ds(cg * rows, rows), :]
            col = jnp.sum(jnp.where(lane128 == j, block, 0.0),
                          axis=1, keepdims=True)
            ge = (cg * CHUNK + rowi * 128 + j).astype(jnp.int32)
            sel = lane128 == i
            cand = jnp.where(sel, col, cand)
            cidx = jnp.where(sel, ge, cidx)
            gm = jnp.where(g_iota == g, NEG, gm)
            return gm, cand, cidx

        cand0 = jnp.full((rows, 128), NEG, jnp.float32)
        cidx0 = jnp.full((rows, 128), IBIG, jnp.int32)
        _, cand, cidx = lax.fori_loop(0, NSEL, gather_one,
                                      (gmax_ref[...], cand0, cidx0))

        def pick_one(i, carry):
            cand, vals, ids = carry
            m = jnp.max(cand)
            idx = jnp.min(jnp.where(cand == m, cidx, IBIG))
            sel = lane128 == i
            vals = jnp.where(sel, m, vals)
            ids = jnp.where(sel, idx, ids)
            cand = jnp.where(cidx == idx, NEG, cand)
            return cand, vals, ids

        vals0 = jnp.zeros((1, 128), jnp.float32)
        ids0 = jnp.zeros((1, 128), jnp.int32)
        _, vals, ids = lax.fori_loop(0, K, pick_one, (cand, vals0, ids0))
        vals_ref[...] = vals
        ids_ref[...] = ids


def kernel(query_emb, index, k):
    del k
    vals, ids = pl.pallas_call(
        lambda q, x, v, i, s, g: _topk_body(q, x, v, i, s, g),
        grid=(NCHUNK,),
        in_specs=[
            pl.BlockSpec((1, D), lambda c: (0, 0)),
            pl.BlockSpec((CHUNK, D), lambda c: (c, 0)),
        ],
        out_specs=[
            pl.BlockSpec((1, 128), lambda c: (0, 0)),
            pl.BlockSpec((1, 128), lambda c: (0, 0)),
        ],
        out_shape=[
            jax.ShapeDtypeStruct((1, 128), jnp.float32),
            jax.ShapeDtypeStruct((1, 128), jnp.int32),
        ],
        scratch_shapes=[
            pltpu.VMEM((NCHUNK * (CHUNK // 128), 128), jnp.float32),
            pltpu.VMEM((128, 128), jnp.float32),
        ],
    )(query_emb, index)
    return ids[0, :K], vals[0, :K]


# speedup vs baseline: 17.3834x; 1.0017x over previous
import jax
import jax.numpy as jnp
from jax import lax
from jax.experimental import pallas as pl
from jax.experimental.pallas import tpu as pltpu

N = 1_000_000
D = 64
CHUNK = 32768
NCHUNK = (N + CHUNK - 1) // CHUNK
ROWS = NCHUNK * (CHUNK // 128)
K = 100
NSEL = 128
NEG = float("-inf")
IBIG = 2**31 - 1


def _topk_body(q_ref, x_ref, vals_ref, ids_ref, scores_ref, gmax_ref):
    c = pl.program_id(0)

    @pl.when(c == 0)
    def _init():
        gmax_ref[...] = jnp.full((128, 128), NEG, jnp.float32)

    x = x_ref[...]
    q = q_ref[...]
    s = lax.dot_general(q, x, (((1,), (1,)), ((), ())),
                        preferred_element_type=jnp.float32)

    @pl.when(c == NCHUNK - 1)
    def _mask_tail():
        eidx = (NCHUNK - 1) * CHUNK + lax.broadcasted_iota(
            jnp.int32, (1, CHUNK), 1)
        sm = jnp.where(eidx < N, s, NEG)
        s2m = sm.reshape(CHUNK // 128, 128)
        scores_ref[pl.ds(c * (CHUNK // 128), CHUNK // 128), :] = s2m
        gmax_ref[pl.ds(c, 1), :] = jnp.max(s2m, axis=0, keepdims=True)

    @pl.when(c != NCHUNK - 1)
    def _store_full():
        s2 = s.reshape(CHUNK // 128, 128)
        scores_ref[pl.ds(c * (CHUNK // 128), CHUNK // 128), :] = s2
        gmax_ref[pl.ds(c, 1), :] = jnp.max(s2, axis=0, keepdims=True)

    @pl.when(c == NCHUNK - 1)
    def _select():
        gm = gmax_ref[...]
        g_iota = (lax.broadcasted_iota(jnp.int32, (128, 128), 0) * 128
                  + lax.broadcasted_iota(jnp.int32, (128, 128), 1))
        lane128 = lax.broadcasted_iota(jnp.int32, (1, 128), 1)
        rows = CHUNK // 128
        rowi = lax.broadcasted_iota(jnp.int32, (rows, 1), 0)

        def gather_one(i, carry):
            gm, cand, cidx = carry
            m = jnp.max(gm)
            g = jnp.min(jnp.where(gm == m, g_iota, IBIG))
            cg = g // 128
            j = g - cg * 128
            block = scores_ref[pl.ds(cg * rows, rows), :]
            col = jnp.sum(jnp.where(lane128 == j, block, 0.0),
                          axis=1, keepdims=True)
            ge = (cg * CHUNK + rowi * 128 + j).astype(jnp.int32)
            sel = lane128 == i
            cand = jnp.where(sel, col, cand)
            cidx = jnp.where(sel, ge, cidx)
            gm = jnp.where(g_iota == g, NEG, gm)
            return gm, cand, cidx

        cand0 = jnp.full((rows, 128), NEG, jnp.float32)
        cidx0 = jnp.full((rows, 128), IBIG, jnp.int32)
        _, cand, cidx = lax.fori_loop(0, NSEL, gather_one,
                                      (gmax_ref[...], cand0, cidx0))

        def pick_one(i, carry):
            cand, vals, ids = carry
            m = jnp.max(cand)
            idx = jnp.min(jnp.where(cand == m, cidx, IBIG))
            sel = lane128 == i
            vals = jnp.where(sel, m, vals)
            ids = jnp.where(sel, idx, ids)
            cand = jnp.where(cidx == idx, NEG, cand)
            return cand, vals, ids

        vals0 = jnp.zeros((1, 128), jnp.float32)
        ids0 = jnp.zeros((1, 128), jnp.int32)
        _, vals, ids = lax.fori_loop(0, K, pick_one, (cand, vals0, ids0))
        vals_ref[...] = vals
        ids_ref[...] = ids


def kernel(query_emb, index, k):
    del k
    vals, ids = pl.pallas_call(
        lambda q, x, v, i, s, g: _topk_body(q, x, v, i, s, g),
        grid=(NCHUNK,),
        in_specs=[
            pl.BlockSpec((1, D), lambda c: (0, 0)),
            pl.BlockSpec((CHUNK, D), lambda c: (c, 0)),
        ],
        out_specs=[
            pl.BlockSpec((1, 128), lambda c: (0, 0)),
            pl.BlockSpec((1, 128), lambda c: (0, 0)),
        ],
        out_shape=[
            jax.ShapeDtypeStruct((1, 128), jnp.float32),
            jax.ShapeDtypeStruct((1, 128), jnp.int32),
        ],
        scratch_shapes=[
            pltpu.VMEM((NCHUNK * (CHUNK // 128), 128), jnp.float32),
            pltpu.VMEM((128, 128), jnp.float32),
        ],
    )(query_emb, index)
    return ids[0, :K], vals[0, :K]


# speedup vs baseline: 18.0048x; 1.0357x over previous
import jax
import jax.numpy as jnp
from jax import lax
from jax.experimental import pallas as pl
from jax.experimental.pallas import tpu as pltpu

N = 1_000_000
D = 64
CHUNK = 32768
NCHUNK = (N + CHUNK - 1) // CHUNK
RPC = CHUNK // 128
ROWS = NCHUNK * RPC
K = 100
NSEL = 128
NEG = float("-inf")
IBIG = 2**31 - 1


def _topk_body(q_ref, x_ref, vals_ref, ids_ref,
               scores_ref, gmax_ref, cand_ref, cidx_ref):
    c = pl.program_id(0)

    @pl.when(c == 0)
    def _init():
        gmax_ref[...] = jnp.full((64, 128), NEG, jnp.float32)

    x = x_ref[...]
    q = q_ref[...]
    s = lax.dot_general(q, x, (((1,), (1,)), ((), ())),
                        preferred_element_type=jnp.float32)

    @pl.when(c == NCHUNK - 1)
    def _mask_store_tail():
        eidx = (NCHUNK - 1) * CHUNK + lax.broadcasted_iota(
            jnp.int32, (1, CHUNK), 1)
        sm = jnp.where(eidx < N, s, NEG)
        s2 = sm.reshape(RPC, 128)
        scores_ref[pl.ds(c * RPC, RPC), :] = s2
        rm = jnp.max(s2, axis=1, keepdims=True)
        gmax_ref[pl.ds(c * (RPC // 128), RPC // 128), :] = rm.reshape(
            RPC // 128, 128)

    @pl.when(c != NCHUNK - 1)
    def _store_full():
        s2 = s.reshape(RPC, 128)
        scores_ref[pl.ds(c * RPC, RPC), :] = s2
        rm = jnp.max(s2, axis=1, keepdims=True)
        gmax_ref[pl.ds(c * (RPC // 128), RPC // 128), :] = rm.reshape(
            RPC // 128, 128)

    @pl.when(c == NCHUNK - 1)
    def _select():
        g_iota = (lax.broadcasted_iota(jnp.int32, (64, 128), 0) * 128
                  + lax.broadcasted_iota(jnp.int32, (64, 128), 1))
        lane128 = lax.broadcasted_iota(jnp.int32, (1, 128), 1)

        def gather_one(i, gm):
            m = jnp.max(gm)
            g = jnp.min(jnp.where(gm == m, g_iota, IBIG))
            cand_ref[pl.ds(i, 1), :] = scores_ref[pl.ds(g, 1), :]
            cidx_ref[pl.ds(i, 1), :] = g * 128 + lane128
            return jnp.where(g_iota == g, NEG, gm)

        lax.fori_loop(0, NSEL, gather_one, gmax_ref[...])

        cidx = cidx_ref[...]

        def pick_one(i, carry):
            cand, vals, ids = carry
            m = jnp.max(cand)
            idx = jnp.min(jnp.where(cand == m, cidx, IBIG))
            sel = lane128 == i
            vals = jnp.where(sel, m, vals)
            ids = jnp.where(sel, idx, ids)
            cand = jnp.where(cidx == idx, NEG, cand)
            return cand, vals, ids

        vals0 = jnp.zeros((1, 128), jnp.float32)
        ids0 = jnp.zeros((1, 128), jnp.int32)
        _, vals, ids = lax.fori_loop(0, K, pick_one,
                                     (cand_ref[...], vals0, ids0))
        vals_ref[...] = vals
        ids_ref[...] = ids


def kernel(query_emb, index, k):
    del k
    vals, ids = pl.pallas_call(
        _topk_body,
        grid=(NCHUNK,),
        in_specs=[
            pl.BlockSpec((1, D), lambda c: (0, 0)),
            pl.BlockSpec((CHUNK, D), lambda c: (c, 0)),
        ],
        out_specs=[
            pl.BlockSpec((1, 128), lambda c: (0, 0)),
            pl.BlockSpec((1, 128), lambda c: (0, 0)),
        ],
        out_shape=[
            jax.ShapeDtypeStruct((1, 128), jnp.float32),
            jax.ShapeDtypeStruct((1, 128), jnp.int32),
        ],
        scratch_shapes=[
            pltpu.VMEM((ROWS, 128), jnp.float32),
            pltpu.VMEM((64, 128), jnp.float32),
            pltpu.VMEM((NSEL, 128), jnp.float32),
            pltpu.VMEM((NSEL, 128), jnp.int32),
        ],
    )(query_emb, index)
    return ids[0, :K], vals[0, :K]


# speedup vs baseline: 18.4020x; 1.0221x over previous
import jax
import jax.numpy as jnp
from jax import lax
from jax.experimental import pallas as pl
from jax.experimental.pallas import tpu as pltpu

N = 1_000_000
D = 64
CHUNK = 32768
NCHUNK = (N + CHUNK - 1) // CHUNK
RPC = CHUNK // 128
ROWS = NCHUNK * RPC
K = 100
NSEL = 128
NEG = float("-inf")
IBIG = 2**31 - 1


def _topk_body(q_ref, x_ref, vals_ref, ids_ref, scores_ref, gmax_ref):
    c = pl.program_id(0)

    @pl.when(c == 0)
    def _init():
        gmax_ref[...] = jnp.full((64, 128), NEG, jnp.float32)

    x = x_ref[...]
    q = q_ref[...]
    s = lax.dot_general(q, x, (((1,), (1,)), ((), ())),
                        preferred_element_type=jnp.float32)

    @pl.when(c == NCHUNK - 1)
    def _mask_store_tail():
        eidx = (NCHUNK - 1) * CHUNK + lax.broadcasted_iota(
            jnp.int32, (1, CHUNK), 1)
        sm = jnp.where(eidx < N, s, NEG)
        s2 = sm.reshape(RPC, 128)
        scores_ref[pl.ds(c * RPC, RPC), :] = s2
        rm = jnp.max(s2, axis=1, keepdims=True)
        gmax_ref[pl.ds(c * (RPC // 128), RPC // 128), :] = rm.reshape(
            RPC // 128, 128)

    @pl.when(c != NCHUNK - 1)
    def _store_full():
        s2 = s.reshape(RPC, 128)
        scores_ref[pl.ds(c * RPC, RPC), :] = s2
        rm = jnp.max(s2, axis=1, keepdims=True)
        gmax_ref[pl.ds(c * (RPC // 128), RPC // 128), :] = rm.reshape(
            RPC // 128, 128)

    @pl.when(c == NCHUNK - 1)
    def _select():
        g_iota = (lax.broadcasted_iota(jnp.int32, (64, 128), 0) * 128
                  + lax.broadcasted_iota(jnp.int32, (64, 128), 1))
        lane128 = lax.broadcasted_iota(jnp.int32, (1, 128), 1)

        def pick(i, carry):
            gm, vals, ids = carry
            m = jnp.max(gm)
            g = jnp.min(jnp.where(gm == m, g_iota, IBIG))
            row = scores_ref[pl.ds(g, 1), :]
            eidx = g * 128 + lane128
            idx = jnp.min(jnp.where(row == m, eidx, IBIG))
            row2 = jnp.where(eidx == idx, NEG, row)
            scores_ref[pl.ds(g, 1), :] = row2
            gm = jnp.where(g_iota == g, jnp.max(row2), gm)
            sel = lane128 == i
            vals = jnp.where(sel, m, vals)
            ids = jnp.where(sel, idx, ids)
            return gm, vals, ids

        vals0 = jnp.zeros((1, 128), jnp.float32)
        ids0 = jnp.zeros((1, 128), jnp.int32)
        _, vals, ids = lax.fori_loop(0, K, pick,
                                     (gmax_ref[...], vals0, ids0))
        vals_ref[...] = vals
        ids_ref[...] = ids


def kernel(query_emb, index, k):
    del k
    vals, ids = pl.pallas_call(
        _topk_body,
        grid=(NCHUNK,),
        in_specs=[
            pl.BlockSpec((1, D), lambda c: (0, 0)),
            pl.BlockSpec((CHUNK, D), lambda c: (c, 0)),
        ],
        out_specs=[
            pl.BlockSpec((1, 128), lambda c: (0, 0)),
            pl.BlockSpec((1, 128), lambda c: (0, 0)),
        ],
        out_shape=[
            jax.ShapeDtypeStruct((1, 128), jnp.float32),
            jax.ShapeDtypeStruct((1, 128), jnp.int32),
        ],
        scratch_shapes=[
            pltpu.VMEM((ROWS, 128), jnp.float32),
            pltpu.VMEM((64, 128), jnp.float32),
        ],
    )(query_emb, index)
    return ids[0, :K], vals[0, :K]


# speedup vs baseline: 18.4102x; 1.0004x over previous
import jax
import jax.numpy as jnp
from jax import lax
from jax.experimental import pallas as pl
from jax.experimental.pallas import tpu as pltpu

N = 1_000_000
D = 64
CHUNK = 32768
NCHUNK = (N + CHUNK - 1) // CHUNK
RPC = CHUNK // 128
ROWS = NCHUNK * RPC
K = 100
NSEL = 128
NEG = float("-inf")
IBIG = 2**31 - 1


def _topk_body(q_ref, x_ref, vals_ref, ids_ref, scores_ref, gmax_ref):
    c = pl.program_id(0)

    @pl.when(c == 0)
    def _init():
        gmax_ref[...] = jnp.full((64, 128), NEG, jnp.float32)

    x = x_ref[...]
    q = q_ref[...]
    s = lax.dot_general(q, x, (((1,), (1,)), ((), ())),
                        preferred_element_type=jnp.float32)

    @pl.when(c == NCHUNK - 1)
    def _mask_store_tail():
        eidx = (NCHUNK - 1) * CHUNK + lax.broadcasted_iota(
            jnp.int32, (1, CHUNK), 1)
        sm = jnp.where(eidx < N, s, NEG)
        s2 = sm.reshape(RPC, 128)
        scores_ref[pl.ds(c * RPC, RPC), :] = s2
        rm = jnp.max(s2, axis=1, keepdims=True)
        gmax_ref[pl.ds(c * (RPC // 128), RPC // 128), :] = rm.reshape(
            RPC // 128, 128)

    @pl.when(c != NCHUNK - 1)
    def _store_full():
        s2 = s.reshape(RPC, 128)
        scores_ref[pl.ds(c * RPC, RPC), :] = s2
        rm = jnp.max(s2, axis=1, keepdims=True)
        gmax_ref[pl.ds(c * (RPC // 128), RPC // 128), :] = rm.reshape(
            RPC // 128, 128)

    @pl.when(c == NCHUNK - 1)
    def _select():
        g_iota = (lax.broadcasted_iota(jnp.int32, (64, 128), 0) * 128
                  + lax.broadcasted_iota(jnp.int32, (64, 128), 1))
        lane128 = lax.broadcasted_iota(jnp.int32, (1, 128), 1)

        def pick(i, carry):
            gm, vals, ids = carry
            m = jnp.max(gm)
            g = jnp.min(jnp.where(gm == m, g_iota, IBIG))
            row = scores_ref[pl.ds(g, 1), :]
            eidx = g * 128 + lane128
            idx = jnp.min(jnp.where(row == m, eidx, IBIG))
            row2 = jnp.where(eidx == idx, NEG, row)
            scores_ref[pl.ds(g, 1), :] = row2
            gm = jnp.where(g_iota == g, jnp.max(row2), gm)
            sel = lane128 == i
            vals = jnp.where(sel, m, vals)
            ids = jnp.where(sel, idx, ids)
            return gm, vals, ids

        vals0 = jnp.zeros((1, 128), jnp.float32)
        ids0 = jnp.zeros((1, 128), jnp.int32)
        _, vals, ids = lax.fori_loop(0, K, pick,
                                     (gmax_ref[...], vals0, ids0),
                                     unroll=4)
        vals_ref[...] = vals
        ids_ref[...] = ids


def kernel(query_emb, index, k):
    del k
    vals, ids = pl.pallas_call(
        _topk_body,
        grid=(NCHUNK,),
        in_specs=[
            pl.BlockSpec((1, D), lambda c: (0, 0)),
            pl.BlockSpec((CHUNK, D), lambda c: (c, 0)),
        ],
        out_specs=[
            pl.BlockSpec((1, 128), lambda c: (0, 0)),
            pl.BlockSpec((1, 128), lambda c: (0, 0)),
        ],
        out_shape=[
            jax.ShapeDtypeStruct((1, 128), jnp.float32),
            jax.ShapeDtypeStruct((1, 128), jnp.int32),
        ],
        scratch_shapes=[
            pltpu.VMEM((ROWS, 128), jnp.float32),
            pltpu.VMEM((64, 128), jnp.float32),
        ],
    )(query_emb, index)
    return ids[0, :K], vals[0, :K]
